# Initial kernel scaffold; baseline (speedup 1.0000x reference)
import jax
import jax.numpy as jnp
from jax.experimental import pallas as pl


def kernel(x, cache_geg):
    raise NotImplementedError("write your pallas kernel here")


# speedup vs baseline: 4.1345x; 4.1345x over previous
import jax
import jax.numpy as jnp
from jax import lax
from jax.experimental import pallas as pl
from jax.experimental.pallas import tpu as pltpu
from jax.experimental.pallas import tpu_sc as plsc

_D = 64
_NC = 2
_NS = 16
_NW = _NC * _NS
_CHUNK = 1024


def _gather_body(x_hbm, table_hbm, out_hbm, idx_v, rows_v, sem):
    wid = lax.axis_index("s") * _NC + lax.axis_index("c")
    n_total = x_hbm.shape[0]
    b_per_w = n_total // _NW
    n_chunks = b_per_w // _CHUNK
    base = wid * b_per_w

    def step(g, carry):
        off = base + g * _CHUNK
        pltpu.sync_copy(x_hbm.at[pl.ds(off, _CHUNK)], idx_v)
        pltpu.async_copy(table_hbm.at[idx_v], rows_v, sem).wait()
        pltpu.sync_copy(rows_v, out_hbm.at[pl.ds(off, _CHUNK)])
        return carry

    lax.fori_loop(0, n_chunks, step, 0)


def kernel(x, cache_geg):
    b, s = x.shape
    flat = x.reshape(-1)
    mesh = plsc.VectorSubcoreMesh(core_axis_name="c", subcore_axis_name="s")
    gather = pl.kernel(
        _gather_body,
        mesh=mesh,
        compiler_params=pltpu.CompilerParams(use_tc_tiling_on_sc=False),
        out_type=jax.ShapeDtypeStruct((b * s, _D), jnp.float32),
        scratch_types=[
            pltpu.VMEM((_CHUNK,), jnp.int32),
            pltpu.VMEM((_CHUNK, _D), jnp.float32),
            pltpu.SemaphoreType.DMA,
        ],
    )
    out = gather(flat, cache_geg)
    return out.reshape(b, s, _D)


# speedup vs baseline: 4.2375x; 1.0249x over previous
import jax
import jax.numpy as jnp
from jax import lax
from jax.experimental import pallas as pl
from jax.experimental.pallas import tpu as pltpu
from jax.experimental.pallas import tpu_sc as plsc

_D = 64
_NC = 2
_NS = 16
_NW = _NC * _NS
_CHUNK = 800


def _gather_body(x_hbm, table_hbm, out_hbm, idx_v, rows0, rows1,
                 gsem0, gsem1, osem0, osem1):
    wid = lax.axis_index("s") * _NC + lax.axis_index("c")
    n_total = x_hbm.shape[0]
    b_per_w = n_total // _NW
    n_chunks = b_per_w // _CHUNK
    n_pairs = n_chunks // 2
    base = wid * b_per_w

    rows = (rows0, rows1)
    gsem = (gsem0, gsem1)
    osem = (osem0, osem1)

    pltpu.sync_copy(x_hbm.at[pl.ds(base, b_per_w)], idx_v)

    def gather_of(g, b):
        return pltpu.make_async_copy(
            table_hbm.at[idx_v.at[pl.ds(g * _CHUNK, _CHUNK)]], rows[b], gsem[b])

    def store_of(g, b):
        return pltpu.make_async_copy(
            rows[b], out_hbm.at[pl.ds(base + g * _CHUNK, _CHUNK)], osem[b])

    gather_of(0, 0).start()

    def pair(p, carry):
        g0 = 2 * p
        gather_of(g0, 0).wait()
        @pl.when(p > 0)
        def _():
            store_of(g0 - 1, 1).wait()
        gather_of(g0 + 1, 1).start()
        store_of(g0, 0).start()
        gather_of(g0 + 1, 1).wait()
        @pl.when(p < n_pairs - 1)
        def _():
            store_of(g0, 0).wait()
            gather_of(g0 + 2, 0).start()
        store_of(g0 + 1, 1).start()
        return carry

    lax.fori_loop(0, n_pairs, pair, 0)
    store_of(n_chunks - 2, 0).wait()
    store_of(n_chunks - 1, 1).wait()


def kernel(x, cache_geg):
    b, s = x.shape
    flat = x.reshape(-1)
    n_total = b * s
    b_per_w = n_total // _NW
    mesh = plsc.VectorSubcoreMesh(core_axis_name="c", subcore_axis_name="s")
    gather = pl.kernel(
        _gather_body,
        mesh=mesh,
        compiler_params=pltpu.CompilerParams(use_tc_tiling_on_sc=False),
        out_type=jax.ShapeDtypeStruct((n_total, _D), jnp.float32),
        scratch_types=[
            pltpu.VMEM((b_per_w,), jnp.int32),
            pltpu.VMEM((_CHUNK, _D), jnp.float32),
            pltpu.VMEM((_CHUNK, _D), jnp.float32),
            pltpu.SemaphoreType.DMA,
            pltpu.SemaphoreType.DMA,
            pltpu.SemaphoreType.DMA,
            pltpu.SemaphoreType.DMA,
        ],
    )
    out = gather(flat, cache_geg)
    return out.reshape(b, s, _D)


# speedup vs baseline: 4.2387x; 1.0003x over previous
import jax
import jax.numpy as jnp
from jax import lax
from jax.experimental import pallas as pl
from jax.experimental.pallas import tpu as pltpu
from jax.experimental.pallas import tpu_sc as plsc

_D = 64
_NC = 2
_NS = 16
_NW = _NC * _NS
_CHUNK = 800
_S = 200
_BPC = _CHUNK // _S


def _gather_body(x_hbm, table_hbm, out_hbm, idx_v, rows0, rows1,
                 gsem0, gsem1, osem0, osem1):
    wid = lax.axis_index("s") * _NC + lax.axis_index("c")
    n_total = x_hbm.shape[0]
    b_per_w = n_total // _NW
    n_chunks = b_per_w // _CHUNK
    n_pairs = n_chunks // 2
    base = wid * b_per_w

    rows = (rows0, rows1)
    gsem = (gsem0, gsem1)
    osem = (osem0, osem1)

    pltpu.sync_copy(x_hbm.at[pl.ds(base, b_per_w)], idx_v)

    def gather_of(g, b):
        return pltpu.make_async_copy(
            table_hbm.at[idx_v.at[pl.ds(g * _CHUNK, _CHUNK)]], rows[b], gsem[b])

    def stores_of(g, b):
        batch0 = (base + g * _CHUNK) // _S
        return [
            pltpu.make_async_copy(
                rows[b].at[pl.ds(j * _S, _S)], out_hbm.at[batch0 + j], osem[b])
            for j in range(_BPC)
        ]

    def start_stores(g, b):
        for c in stores_of(g, b):
            c.start()

    def wait_stores(g, b):
        for c in stores_of(g, b):
            c.wait()

    gather_of(0, 0).start()

    def pair(p, carry):
        g0 = 2 * p
        gather_of(g0, 0).wait()
        @pl.when(p > 0)
        def _():
            wait_stores(g0 - 1, 1)
        gather_of(g0 + 1, 1).start()
        start_stores(g0, 0)
        gather_of(g0 + 1, 1).wait()
        @pl.when(p < n_pairs - 1)
        def _():
            wait_stores(g0, 0)
            gather_of(g0 + 2, 0).start()
        start_stores(g0 + 1, 1)
        return carry

    lax.fori_loop(0, n_pairs, pair, 0)
    wait_stores(n_chunks - 2, 0)
    wait_stores(n_chunks - 1, 1)


def kernel(x, cache_geg):
    b, s = x.shape
    flat = x.reshape(-1)
    n_total = b * s
    b_per_w = n_total // _NW
    mesh = plsc.VectorSubcoreMesh(core_axis_name="c", subcore_axis_name="s")
    gather = pl.kernel(
        _gather_body,
        mesh=mesh,
        compiler_params=pltpu.CompilerParams(use_tc_tiling_on_sc=False),
        out_type=jax.ShapeDtypeStruct((b, s, _D), jnp.float32),
        scratch_types=[
            pltpu.VMEM((b_per_w,), jnp.int32),
            pltpu.VMEM((_CHUNK, _D), jnp.float32),
            pltpu.VMEM((_CHUNK, _D), jnp.float32),
            pltpu.SemaphoreType.DMA,
            pltpu.SemaphoreType.DMA,
            pltpu.SemaphoreType.DMA,
            pltpu.SemaphoreType.DMA,
        ],
    )
    return gather(flat, cache_geg)
